# Initial kernel scaffold; baseline (speedup 1.0000x reference)
import jax
import jax.numpy as jnp
from jax.experimental import pallas as pl


def kernel(values, lengths, count):
    raise NotImplementedError("write your pallas kernel here")


# speedup vs baseline: 33.5509x; 33.5509x over previous
import functools

import jax
import jax.numpy as jnp
from jax import lax
from jax.experimental import pallas as pl
from jax.experimental.pallas import tpu as pltpu
from jax.experimental.pallas import tpu_sc as plsc

_N = 3_276_800
_V = 1_000_000
_LANE = 128
_ROWS = _N // _LANE
_NC = 2
_NS = 16
_ROWS_PER_TILE = _ROWS // (_NC * _NS)
_RB = 8
_CHUNKS = _ROWS_PER_TILE // _RB
_SPAN = 62_496
_TAIL = _V - _NS * _SPAN


def _hist_sc(values2d, count):
    mesh = plsc.VectorSubcoreMesh(core_axis_name="c", subcore_axis_name="s")

    @functools.partial(
        pl.kernel,
        mesh=mesh,
        out_type=(
            jax.ShapeDtypeStruct((_V,), jnp.float32),
            jax.ShapeDtypeStruct((_V,), jnp.float32),
        ),
        scratch_types=[
            pltpu.VMEM((_RB, _LANE), jnp.int32),
            pltpu.VMEM((_LANE,), jnp.float32),
            pltpu.VMEM((_SPAN,), jnp.float32),
            pltpu.VMEM_SHARED((_V,), jnp.float32),
            pltpu.SemaphoreType.DMA,
        ],
    )
    def hist(values_hbm, count_hbm, p0_hbm, p1_hbm, vbuf, ones_v, bounce, acc, sem):
        c = lax.axis_index("c")
        s = lax.axis_index("s")

        for i in range(_LANE // 16):
            ones_v[pl.ds(16 * i, 16)] = jnp.ones((16,), jnp.float32)

        off = s * _SPAN
        pltpu.sync_copy(count_hbm.at[pl.ds(off, _SPAN)], bounce)
        pltpu.sync_copy(bounce, acc.at[pl.ds(off, _SPAN)])

        @pl.when(s == _NS - 1)
        def _():
            pltpu.sync_copy(
                count_hbm.at[pl.ds(_NS * _SPAN, _TAIL)], bounce.at[pl.ds(0, _TAIL)]
            )
            pltpu.sync_copy(
                bounce.at[pl.ds(0, _TAIL)], acc.at[pl.ds(_NS * _SPAN, _TAIL)]
            )

        plsc.subcore_barrier()

        row0 = (c * _NS + s) * _ROWS_PER_TILE

        def chunk(g, carry):
            pltpu.sync_copy(values_hbm.at[pl.ds(row0 + g * _RB, _RB)], vbuf)
            handles = []
            for j in range(_RB):
                handles.append(
                    pltpu.async_copy(ones_v, acc.at[vbuf.at[j]], sem, add=True)
                )
            for h in handles:
                h.wait()
            return carry

        lax.fori_loop(0, _CHUNKS, chunk, 0)

        plsc.subcore_barrier()

        pltpu.sync_copy(acc.at[pl.ds(off, _SPAN)], bounce)

        @pl.when(c == 0)
        def _():
            pltpu.sync_copy(bounce, p0_hbm.at[pl.ds(off, _SPAN)])

        @pl.when(c == 1)
        def _():
            pltpu.sync_copy(bounce, p1_hbm.at[pl.ds(off, _SPAN)])

        @pl.when(s == _NS - 1)
        def _():
            pltpu.sync_copy(
                acc.at[pl.ds(_NS * _SPAN, _TAIL)], bounce.at[pl.ds(0, _TAIL)]
            )

            @pl.when(c == 0)
            def _():
                pltpu.sync_copy(
                    bounce.at[pl.ds(0, _TAIL)], p0_hbm.at[pl.ds(_NS * _SPAN, _TAIL)]
                )

            @pl.when(c == 1)
            def _():
                pltpu.sync_copy(
                    bounce.at[pl.ds(0, _TAIL)], p1_hbm.at[pl.ds(_NS * _SPAN, _TAIL)]
                )

    return hist(values2d, count)


def _merge_tc(p0, p1, count):
    a = p0.reshape(800, 1250)
    b = p1.reshape(800, 1250)
    cnt = count.reshape(800, 1250)

    def mk(a_ref, b_ref, c_ref, o_ref):
        o_ref[...] = a_ref[...] + b_ref[...] - c_ref[...]

    out = pl.pallas_call(
        mk,
        out_shape=jax.ShapeDtypeStruct((800, 1250), jnp.float32),
        grid=(10,),
        in_specs=[pl.BlockSpec((80, 1250), lambda i: (i, 0))] * 3,
        out_specs=pl.BlockSpec((80, 1250), lambda i: (i, 0)),
    )(a, b, cnt)
    return out.reshape(_V)


def kernel(values, lengths, count):
    v2d = values.reshape(_ROWS, _LANE)
    p0, p1 = _hist_sc(v2d, count)
    new_count = _merge_tc(p0, p1, count)
    return (values, lengths, new_count)


# speedup vs baseline: 51.9354x; 1.5480x over previous
import functools

import jax
import jax.numpy as jnp
from jax import lax
from jax.experimental import pallas as pl
from jax.experimental.pallas import tpu as pltpu
from jax.experimental.pallas import tpu_sc as plsc

_N = 3_276_800
_V = 1_000_000
_LANE = 128
_ROWS = _N // _LANE
_NC = 2
_NS = 16
_ROWS_PER_TILE = _ROWS // (_NC * _NS)
_RB = 8
_CHUNKS = _ROWS_PER_TILE // _RB
_SPAN = 62_496
_TAIL = _V - _NS * _SPAN


def _hist_sc(values2d, count):
    mesh = plsc.VectorSubcoreMesh(core_axis_name="c", subcore_axis_name="s")

    @functools.partial(
        pl.kernel,
        mesh=mesh,
        out_type=(
            jax.ShapeDtypeStruct((_ROWS, _LANE), jnp.int32),
            jax.ShapeDtypeStruct((_V,), jnp.float32),
            jax.ShapeDtypeStruct((_V,), jnp.float32),
        ),
        scratch_types=[
            pltpu.VMEM((2, _RB, _LANE), jnp.int32),
            pltpu.VMEM((_LANE,), jnp.float32),
            pltpu.VMEM((_SPAN,), jnp.float32),
            pltpu.VMEM_SHARED((_V,), jnp.float32),
            pltpu.SemaphoreType.DMA,
            pltpu.SemaphoreType.DMA,
            pltpu.SemaphoreType.DMA,
        ],
    )
    def hist(values_hbm, count_hbm, vout_hbm, p0_hbm, p1_hbm,
             vbuf, ones_v, bounce, acc, sem_load, sem_scat, sem_out):
        c = lax.axis_index("c")
        s = lax.axis_index("s")

        for i in range(_LANE // 16):
            ones_v[pl.ds(16 * i, 16)] = jnp.ones((16,), jnp.float32)

        off = s * _SPAN
        pltpu.sync_copy(count_hbm.at[pl.ds(off, _SPAN)], bounce)
        pltpu.sync_copy(bounce, acc.at[pl.ds(off, _SPAN)])

        @pl.when(s == _NS - 1)
        def _():
            pltpu.sync_copy(
                count_hbm.at[pl.ds(_NS * _SPAN, _TAIL)], bounce.at[pl.ds(0, _TAIL)]
            )
            pltpu.sync_copy(
                bounce.at[pl.ds(0, _TAIL)], acc.at[pl.ds(_NS * _SPAN, _TAIL)]
            )

        plsc.subcore_barrier()

        row0 = (c * _NS + s) * _ROWS_PER_TILE

        def _load(g, bank):
            return pltpu.make_async_copy(
                values_hbm.at[pl.ds(row0 + g * _RB, _RB)], vbuf.at[bank], sem_load
            )

        def _scat(bank, j):
            return pltpu.make_async_copy(
                ones_v, acc.at[vbuf.at[bank].at[j]], sem_scat
            )

        def _wback(g, bank):
            return pltpu.make_async_copy(
                vbuf.at[bank], vout_hbm.at[pl.ds(row0 + g * _RB, _RB)], sem_out
            )

        _load(0, 0).start()

        def chunk(g, carry):
            b = g & 1
            _load(g, b).wait()

            @pl.when(g > 0)
            def _():
                for j in range(_RB):
                    _scat(1 - b, j).wait()
                _wback(g - 1, 1 - b).wait()

            @pl.when(g < _CHUNKS - 1)
            def _():
                _load(g + 1, 1 - b).start()

            for j in range(_RB):
                pltpu.async_copy(ones_v, acc.at[vbuf.at[b].at[j]], sem_scat,
                                 add=True)
            pltpu.async_copy(vbuf.at[b], vout_hbm.at[pl.ds(row0 + g * _RB, _RB)],
                             sem_out)
            return carry

        lax.fori_loop(0, _CHUNKS, chunk, 0)

        last_b = (_CHUNKS - 1) & 1
        for j in range(_RB):
            _scat(last_b, j).wait()
        _wback(_CHUNKS - 1, last_b).wait()

        plsc.subcore_barrier()

        pltpu.sync_copy(acc.at[pl.ds(off, _SPAN)], bounce)

        @pl.when(c == 0)
        def _():
            pltpu.sync_copy(bounce, p0_hbm.at[pl.ds(off, _SPAN)])

        @pl.when(c == 1)
        def _():
            pltpu.sync_copy(bounce, p1_hbm.at[pl.ds(off, _SPAN)])

        @pl.when(s == _NS - 1)
        def _():
            pltpu.sync_copy(
                acc.at[pl.ds(_NS * _SPAN, _TAIL)], bounce.at[pl.ds(0, _TAIL)]
            )

            @pl.when(c == 0)
            def _():
                pltpu.sync_copy(
                    bounce.at[pl.ds(0, _TAIL)], p0_hbm.at[pl.ds(_NS * _SPAN, _TAIL)]
                )

            @pl.when(c == 1)
            def _():
                pltpu.sync_copy(
                    bounce.at[pl.ds(0, _TAIL)], p1_hbm.at[pl.ds(_NS * _SPAN, _TAIL)]
                )

    return hist(values2d, count)


def _merge_tc(p0, p1, count):
    blk = 102_400

    def mk(a_ref, b_ref, c_ref, o_ref):
        o_ref[...] = a_ref[...] + b_ref[...] - c_ref[...]

    return pl.pallas_call(
        mk,
        out_shape=jax.ShapeDtypeStruct((_V,), jnp.float32),
        grid=(pl.cdiv(_V, blk),),
        in_specs=[pl.BlockSpec((blk,), lambda i: (i,))] * 3,
        out_specs=pl.BlockSpec((blk,), lambda i: (i,)),
    )(p0, p1, count)


def kernel(values, lengths, count):
    v2d = values.reshape(_ROWS, _LANE)
    vout, p0, p1 = _hist_sc(v2d, count)
    new_count = _merge_tc(p0, p1, count)
    return (vout.reshape(_N), lengths, new_count)


# speedup vs baseline: 52.2845x; 1.0067x over previous
import functools

import jax
import jax.numpy as jnp
from jax import lax
from jax.experimental import pallas as pl
from jax.experimental.pallas import tpu as pltpu
from jax.experimental.pallas import tpu_sc as plsc

_N = 3_276_800
_V = 1_000_000
_LANE = 128
_ROWS = _N // _LANE
_NC = 2
_NS = 16
_ROWS_PER_TILE = _ROWS // (_NC * _NS)
_RB = 8
_CHUNKS = _ROWS_PER_TILE // _RB
_SPAN = 62_496
_QCH = _SPAN // 4
_TAIL = _V - _NS * _SPAN


def _hist_sc(values2d, count):
    mesh = plsc.VectorSubcoreMesh(core_axis_name="c", subcore_axis_name="s")

    @functools.partial(
        pl.kernel,
        mesh=mesh,
        out_type=(
            jax.ShapeDtypeStruct((_ROWS, _LANE), jnp.int32),
            jax.ShapeDtypeStruct((_V,), jnp.float32),
            jax.ShapeDtypeStruct((_V,), jnp.float32),
        ),
        scratch_types=[
            pltpu.VMEM((2, _RB, _LANE), jnp.int32),
            pltpu.VMEM((_LANE,), jnp.float32),
            pltpu.VMEM((_QCH,), jnp.float32),
            pltpu.VMEM((_QCH,), jnp.float32),
            pltpu.VMEM_SHARED((_V,), jnp.float32),
            pltpu.SemaphoreType.DMA,
            pltpu.SemaphoreType.DMA,
            pltpu.SemaphoreType.DMA,
            pltpu.SemaphoreType.DMA,
            pltpu.SemaphoreType.DMA,
        ],
    )
    def hist(values_hbm, count_hbm, vout_hbm, p0_hbm, p1_hbm,
             vbuf, ones_v, bounce0, bounce1, acc, sem_load, sem_scat,
             sem_scat2, sem_out, sem_b):
        c = lax.axis_index("c")
        s = lax.axis_index("s")

        for i in range(_LANE // 16):
            ones_v[pl.ds(16 * i, 16)] = jnp.ones((16,), jnp.float32)

        off = s * _SPAN
        banks = (bounce0, bounce1)
        for q in range(4):
            qoff = off + q * _QCH
            pltpu.async_copy(count_hbm.at[pl.ds(qoff, _QCH)], banks[q & 1],
                             sem_b)
            if q > 0:
                pltpu.make_async_copy(
                    banks[1 - (q & 1)],
                    acc.at[pl.ds(off + (q - 1) * _QCH, _QCH)], sem_out,
                ).wait()
            pltpu.make_async_copy(
                count_hbm.at[pl.ds(qoff, _QCH)], banks[q & 1], sem_b
            ).wait()
            pltpu.async_copy(banks[q & 1], acc.at[pl.ds(qoff, _QCH)],
                             sem_out)
        pltpu.make_async_copy(
            banks[1], acc.at[pl.ds(off + 3 * _QCH, _QCH)], sem_out
        ).wait()

        @pl.when(s == _NS - 1)
        def _():
            pltpu.sync_copy(
                count_hbm.at[pl.ds(_NS * _SPAN, _TAIL)],
                bounce0.at[pl.ds(0, _TAIL)],
            )
            pltpu.sync_copy(
                bounce0.at[pl.ds(0, _TAIL)], acc.at[pl.ds(_NS * _SPAN, _TAIL)]
            )

        plsc.subcore_barrier()

        row0 = (c * _NS + s) * _ROWS_PER_TILE

        def _load(g, bank):
            return pltpu.make_async_copy(
                values_hbm.at[pl.ds(row0 + g * _RB, _RB)], vbuf.at[bank], sem_load
            )

        def _scat(bank, j):
            sem = sem_scat if j % 2 == 0 else sem_scat2
            return pltpu.make_async_copy(
                ones_v, acc.at[vbuf.at[bank].at[j]], sem
            )

        def _wback(g, bank):
            return pltpu.make_async_copy(
                vbuf.at[bank], vout_hbm.at[pl.ds(row0 + g * _RB, _RB)], sem_out
            )

        _load(0, 0).start()

        def chunk(g, carry):
            b = g & 1
            _load(g, b).wait()

            @pl.when(g > 0)
            def _():
                for j in range(_RB):
                    _scat(1 - b, j).wait()
                _wback(g - 1, 1 - b).wait()

            @pl.when(g < _CHUNKS - 1)
            def _():
                _load(g + 1, 1 - b).start()

            for j in range(_RB):
                pltpu.async_copy(ones_v, acc.at[vbuf.at[b].at[j]],
                                 sem_scat if j % 2 == 0 else sem_scat2,
                                 add=True)
            pltpu.async_copy(vbuf.at[b], vout_hbm.at[pl.ds(row0 + g * _RB, _RB)],
                             sem_out)
            return carry

        lax.fori_loop(0, _CHUNKS, chunk, 0)

        last_b = (_CHUNKS - 1) & 1
        for j in range(_RB):
            _scat(last_b, j).wait()
        _wback(_CHUNKS - 1, last_b).wait()

        plsc.subcore_barrier()

        def _wout(q, p_hbm):
            qoff = off + q * _QCH
            pltpu.async_copy(acc.at[pl.ds(qoff, _QCH)], banks[q & 1], sem_b)
            if q > 0:
                pltpu.make_async_copy(
                    banks[1 - (q & 1)],
                    p_hbm.at[pl.ds(off + (q - 1) * _QCH, _QCH)], sem_out,
                ).wait()
            pltpu.make_async_copy(
                acc.at[pl.ds(qoff, _QCH)], banks[q & 1], sem_b
            ).wait()
            pltpu.async_copy(banks[q & 1], p_hbm.at[pl.ds(qoff, _QCH)],
                             sem_out)

        @pl.when(c == 0)
        def _():
            for q in range(4):
                _wout(q, p0_hbm)
            pltpu.make_async_copy(
                banks[1], p0_hbm.at[pl.ds(off + 3 * _QCH, _QCH)], sem_out
            ).wait()

        @pl.when(c == 1)
        def _():
            for q in range(4):
                _wout(q, p1_hbm)
            pltpu.make_async_copy(
                banks[1], p1_hbm.at[pl.ds(off + 3 * _QCH, _QCH)], sem_out
            ).wait()

        @pl.when(s == _NS - 1)
        def _():
            pltpu.sync_copy(
                acc.at[pl.ds(_NS * _SPAN, _TAIL)], bounce0.at[pl.ds(0, _TAIL)]
            )

            @pl.when(c == 0)
            def _():
                pltpu.sync_copy(
                    bounce0.at[pl.ds(0, _TAIL)],
                    p0_hbm.at[pl.ds(_NS * _SPAN, _TAIL)],
                )

            @pl.when(c == 1)
            def _():
                pltpu.sync_copy(
                    bounce0.at[pl.ds(0, _TAIL)],
                    p1_hbm.at[pl.ds(_NS * _SPAN, _TAIL)],
                )

    return hist(values2d, count)


def _merge_tc(p0, p1, count):
    blk = 102_400

    def mk(a_ref, b_ref, c_ref, o_ref):
        o_ref[...] = a_ref[...] + b_ref[...] - c_ref[...]

    return pl.pallas_call(
        mk,
        out_shape=jax.ShapeDtypeStruct((_V,), jnp.float32),
        grid=(pl.cdiv(_V, blk),),
        in_specs=[pl.BlockSpec((blk,), lambda i: (i,))] * 3,
        out_specs=pl.BlockSpec((blk,), lambda i: (i,)),
    )(p0, p1, count)


def kernel(values, lengths, count):
    v2d = values.reshape(_ROWS, _LANE)
    vout, p0, p1 = _hist_sc(v2d, count)
    new_count = _merge_tc(p0, p1, count)
    return (vout.reshape(_N), lengths, new_count)


# speedup vs baseline: 53.1655x; 1.0168x over previous
import functools

import jax
import jax.numpy as jnp
from jax import lax
from jax.experimental import pallas as pl
from jax.experimental.pallas import tpu as pltpu
from jax.experimental.pallas import tpu_sc as plsc

_N = 3_276_800
_V = 1_000_000
_LANE = 128
_ROWS = _N // _LANE
_NC = 2
_NS = 16
_ROWS_PER_TILE = _ROWS // (_NC * _NS)
_CW = 1_024
_VALS_PER_TILE = _N // (_NC * _NS)
_CHUNKS = _VALS_PER_TILE // _CW
_SPAN = 62_496
_QCH = _SPAN // 4
_TAIL = _V - _NS * _SPAN


def _hist_sc(values2d, count):
    mesh = plsc.VectorSubcoreMesh(core_axis_name="c", subcore_axis_name="s")

    @functools.partial(
        pl.kernel,
        mesh=mesh,
        out_type=(
            jax.ShapeDtypeStruct((_N,), jnp.int32),
            jax.ShapeDtypeStruct((_V,), jnp.float32),
            jax.ShapeDtypeStruct((_V,), jnp.float32),
        ),
        scratch_types=[
            pltpu.VMEM((_CW,), jnp.int32),
            pltpu.VMEM((_CW,), jnp.int32),
            pltpu.VMEM((_CW,), jnp.float32),
            pltpu.VMEM((_QCH,), jnp.float32),
            pltpu.VMEM((_QCH,), jnp.float32),
            pltpu.VMEM_SHARED((_V,), jnp.float32),
            pltpu.SemaphoreType.DMA,
            pltpu.SemaphoreType.DMA,
            pltpu.SemaphoreType.DMA,
            pltpu.SemaphoreType.DMA,
            pltpu.SemaphoreType.DMA,
        ],
    )
    def hist(values_hbm, count_hbm, vout_hbm, p0_hbm, p1_hbm,
             vbufA, vbufB, ones_v, bounce0, bounce1, acc, sem_load, sem_scat,
             sem_scat2, sem_out, sem_b):
        c = lax.axis_index("c")
        s = lax.axis_index("s")

        for i in range(_CW // 16):
            ones_v[pl.ds(16 * i, 16)] = jnp.ones((16,), jnp.float32)

        off = s * _SPAN
        banks = (bounce0, bounce1)
        for q in range(4):
            qoff = off + q * _QCH
            pltpu.async_copy(count_hbm.at[pl.ds(qoff, _QCH)], banks[q & 1],
                             sem_b)
            if q > 0:
                pltpu.make_async_copy(
                    banks[1 - (q & 1)],
                    acc.at[pl.ds(off + (q - 1) * _QCH, _QCH)], sem_out,
                ).wait()
            pltpu.make_async_copy(
                count_hbm.at[pl.ds(qoff, _QCH)], banks[q & 1], sem_b
            ).wait()
            pltpu.async_copy(banks[q & 1], acc.at[pl.ds(qoff, _QCH)],
                             sem_out)
        pltpu.make_async_copy(
            banks[1], acc.at[pl.ds(off + 3 * _QCH, _QCH)], sem_out
        ).wait()

        @pl.when(s == _NS - 1)
        def _():
            pltpu.sync_copy(
                count_hbm.at[pl.ds(_NS * _SPAN, _TAIL)],
                bounce0.at[pl.ds(0, _TAIL)],
            )
            pltpu.sync_copy(
                bounce0.at[pl.ds(0, _TAIL)], acc.at[pl.ds(_NS * _SPAN, _TAIL)]
            )

        plsc.subcore_barrier()

        base0 = (c * _NS + s) * _VALS_PER_TILE
        vbanks = (vbufA, vbufB)

        def _load(g, bank):
            return pltpu.make_async_copy(
                values_hbm.at[pl.ds(base0 + g * _CW, _CW)], vbanks[bank],
                sem_load,
            )

        def _scat(bank):
            return pltpu.make_async_copy(
                ones_v, acc.at[vbanks[bank]],
                sem_scat if bank == 0 else sem_scat2,
            )

        def _wback(g, bank):
            return pltpu.make_async_copy(
                vbanks[bank], vout_hbm.at[pl.ds(base0 + g * _CW, _CW)], sem_out
            )

        _load(0, 0).start()

        def chunk(h, carry):
            for b in range(2):
                g = 2 * h + b
                _load(g, b).wait()

                @pl.when(g > 0)
                def _():
                    _scat(1 - b).wait()
                    _wback(g - 1, 1 - b).wait()

                @pl.when(g < _CHUNKS - 1)
                def _():
                    _load(g + 1, 1 - b).start()

                pltpu.async_copy(
                    ones_v, acc.at[vbanks[b]],
                    sem_scat if b == 0 else sem_scat2, add=True,
                )
                pltpu.async_copy(
                    vbanks[b], vout_hbm.at[pl.ds(base0 + g * _CW, _CW)], sem_out
                )
            return carry

        lax.fori_loop(0, _CHUNKS // 2, chunk, 0)

        _scat(1).wait()
        _wback(_CHUNKS - 1, 1).wait()

        plsc.subcore_barrier()

        def _wout(q, p_hbm):
            qoff = off + q * _QCH
            pltpu.async_copy(acc.at[pl.ds(qoff, _QCH)], banks[q & 1], sem_b)
            if q > 0:
                pltpu.make_async_copy(
                    banks[1 - (q & 1)],
                    p_hbm.at[pl.ds(off + (q - 1) * _QCH, _QCH)], sem_out,
                ).wait()
            pltpu.make_async_copy(
                acc.at[pl.ds(qoff, _QCH)], banks[q & 1], sem_b
            ).wait()
            pltpu.async_copy(banks[q & 1], p_hbm.at[pl.ds(qoff, _QCH)],
                             sem_out)

        @pl.when(c == 0)
        def _():
            for q in range(4):
                _wout(q, p0_hbm)
            pltpu.make_async_copy(
                banks[1], p0_hbm.at[pl.ds(off + 3 * _QCH, _QCH)], sem_out
            ).wait()

        @pl.when(c == 1)
        def _():
            for q in range(4):
                _wout(q, p1_hbm)
            pltpu.make_async_copy(
                banks[1], p1_hbm.at[pl.ds(off + 3 * _QCH, _QCH)], sem_out
            ).wait()

        @pl.when(s == _NS - 1)
        def _():
            pltpu.sync_copy(
                acc.at[pl.ds(_NS * _SPAN, _TAIL)], bounce0.at[pl.ds(0, _TAIL)]
            )

            @pl.when(c == 0)
            def _():
                pltpu.sync_copy(
                    bounce0.at[pl.ds(0, _TAIL)],
                    p0_hbm.at[pl.ds(_NS * _SPAN, _TAIL)],
                )

            @pl.when(c == 1)
            def _():
                pltpu.sync_copy(
                    bounce0.at[pl.ds(0, _TAIL)],
                    p1_hbm.at[pl.ds(_NS * _SPAN, _TAIL)],
                )

    return hist(values2d, count)


def _merge_tc(p0, p1, count):
    blk = 102_400

    def mk(a_ref, b_ref, c_ref, o_ref):
        o_ref[...] = a_ref[...] + b_ref[...] - c_ref[...]

    return pl.pallas_call(
        mk,
        out_shape=jax.ShapeDtypeStruct((_V,), jnp.float32),
        grid=(pl.cdiv(_V, blk),),
        in_specs=[pl.BlockSpec((blk,), lambda i: (i,))] * 3,
        out_specs=pl.BlockSpec((blk,), lambda i: (i,)),
    )(p0, p1, count)


def kernel(values, lengths, count):
    vout, p0, p1 = _hist_sc(values, count)
    new_count = _merge_tc(p0, p1, count)
    return (vout, lengths, new_count)


# speedup vs baseline: 54.2102x; 1.0196x over previous
import functools

import jax
import jax.numpy as jnp
from jax import lax
from jax.experimental import pallas as pl
from jax.experimental.pallas import tpu as pltpu
from jax.experimental.pallas import tpu_sc as plsc

_N = 3_276_800
_V = 1_000_000
_LANE = 128
_ROWS = _N // _LANE
_NC = 2
_NS = 16
_ROWS_PER_TILE = _ROWS // (_NC * _NS)
_CW = 1_024
_VALS_PER_TILE = _N // (_NC * _NS)
_CHUNKS = _VALS_PER_TILE // _CW
_SPAN = 62_496
_QCH = _SPAN // 4
_TAIL = _V - _NS * _SPAN


def _hist_sc(values2d, count):
    mesh = plsc.VectorSubcoreMesh(core_axis_name="c", subcore_axis_name="s")

    @functools.partial(
        pl.kernel,
        mesh=mesh,
        out_type=(
            jax.ShapeDtypeStruct((_N,), jnp.int32),
            jax.ShapeDtypeStruct((_V,), jnp.float32),
            jax.ShapeDtypeStruct((_V,), jnp.float32),
        ),
        scratch_types=[
            pltpu.VMEM((_CW,), jnp.int32),
            pltpu.VMEM((_CW,), jnp.int32),
            pltpu.VMEM((_CW,), jnp.float32),
            pltpu.VMEM((_QCH,), jnp.float32),
            pltpu.VMEM((_QCH,), jnp.float32),
            pltpu.VMEM((_QCH,), jnp.float32),
            pltpu.VMEM((_QCH,), jnp.float32),
            pltpu.VMEM_SHARED((_V,), jnp.float32),
            pltpu.SemaphoreType.DMA,
            pltpu.SemaphoreType.DMA,
            pltpu.SemaphoreType.DMA,
            pltpu.SemaphoreType.DMA,
            pltpu.SemaphoreType.DMA,
        ],
    )
    def hist(values_hbm, count_hbm, vout_hbm, p0_hbm, p1_hbm,
             vbufA, vbufB, ones_v, bounce0, bounce1, bounce2, bounce3, acc,
             sem_load, sem_scat, sem_scat2, sem_out, sem_b):
        c = lax.axis_index("c")
        s = lax.axis_index("s")

        base0 = (c * _NS + s) * _VALS_PER_TILE
        pltpu.async_copy(values_hbm.at[pl.ds(base0, _CW)], vbufA, sem_load)

        for i in range(_CW // 16):
            ones_v[pl.ds(16 * i, 16)] = jnp.ones((16,), jnp.float32)

        off = s * _SPAN
        banks = (bounce0, bounce1, bounce2, bounce3)

        @pl.when(c == 0)
        def _():
            for q in range(4):
                pltpu.async_copy(
                    count_hbm.at[pl.ds(off + q * _QCH, _QCH)], banks[q], sem_b
                )
            for q in range(4):
                pltpu.make_async_copy(
                    count_hbm.at[pl.ds(off + q * _QCH, _QCH)], banks[q], sem_b
                ).wait()
                pltpu.async_copy(
                    banks[q], acc.at[pl.ds(off + q * _QCH, _QCH)], sem_out
                )
            for q in range(4):
                pltpu.make_async_copy(
                    banks[q], acc.at[pl.ds(off + q * _QCH, _QCH)], sem_out
                ).wait()

            @pl.when(s == _NS - 1)
            def _():
                pltpu.sync_copy(
                    count_hbm.at[pl.ds(_NS * _SPAN, _TAIL)],
                    bounce0.at[pl.ds(0, _TAIL)],
                )
                pltpu.sync_copy(
                    bounce0.at[pl.ds(0, _TAIL)],
                    acc.at[pl.ds(_NS * _SPAN, _TAIL)],
                )

        @pl.when(c == 1)
        def _():
            zero = jnp.zeros((16,), jnp.float32)

            def zfill(i, carry):
                bounce0[pl.ds(i * 16, 16)] = zero
                return carry

            lax.fori_loop(0, _QCH // 16, zfill, 0)
            bounce0[pl.ds(_QCH - 16, 16)] = zero
            for q in range(4):
                pltpu.async_copy(
                    bounce0, acc.at[pl.ds(off + q * _QCH, _QCH)], sem_out
                )
            for q in range(4):
                pltpu.make_async_copy(
                    bounce0, acc.at[pl.ds(off + q * _QCH, _QCH)], sem_out
                ).wait()

            @pl.when(s == _NS - 1)
            def _():
                pltpu.sync_copy(
                    bounce0.at[pl.ds(0, _TAIL)],
                    acc.at[pl.ds(_NS * _SPAN, _TAIL)],
                )

        plsc.subcore_barrier()

        vbanks = (vbufA, vbufB)

        def _load(g, bank):
            return pltpu.make_async_copy(
                values_hbm.at[pl.ds(base0 + g * _CW, _CW)], vbanks[bank],
                sem_load,
            )

        def _scat(bank):
            return pltpu.make_async_copy(
                ones_v, acc.at[vbanks[bank]],
                sem_scat if bank == 0 else sem_scat2,
            )

        def _wback(g, bank):
            return pltpu.make_async_copy(
                vbanks[bank], vout_hbm.at[pl.ds(base0 + g * _CW, _CW)], sem_out
            )

        def chunk(h, carry):
            for b in range(2):
                g = 2 * h + b
                _load(g, b).wait()

                @pl.when(g > 0)
                def _():
                    _scat(1 - b).wait()
                    _wback(g - 1, 1 - b).wait()

                @pl.when(g < _CHUNKS - 1)
                def _():
                    _load(g + 1, 1 - b).start()

                pltpu.async_copy(
                    ones_v, acc.at[vbanks[b]],
                    sem_scat if b == 0 else sem_scat2, add=True,
                )
                pltpu.async_copy(
                    vbanks[b], vout_hbm.at[pl.ds(base0 + g * _CW, _CW)], sem_out
                )
            return carry

        lax.fori_loop(0, _CHUNKS // 2, chunk, 0)

        _scat(1).wait()
        _wback(_CHUNKS - 1, 1).wait()

        plsc.subcore_barrier()

        def _wout(p_hbm):
            for q in range(4):
                pltpu.async_copy(
                    acc.at[pl.ds(off + q * _QCH, _QCH)], banks[q], sem_b
                )
            for q in range(4):
                pltpu.make_async_copy(
                    acc.at[pl.ds(off + q * _QCH, _QCH)], banks[q], sem_b
                ).wait()
                pltpu.async_copy(
                    banks[q], p_hbm.at[pl.ds(off + q * _QCH, _QCH)], sem_out
                )
            for q in range(4):
                pltpu.make_async_copy(
                    banks[q], p_hbm.at[pl.ds(off + q * _QCH, _QCH)], sem_out
                ).wait()

            @pl.when(s == _NS - 1)
            def _():
                pltpu.sync_copy(
                    acc.at[pl.ds(_NS * _SPAN, _TAIL)],
                    bounce0.at[pl.ds(0, _TAIL)],
                )
                pltpu.sync_copy(
                    bounce0.at[pl.ds(0, _TAIL)],
                    p_hbm.at[pl.ds(_NS * _SPAN, _TAIL)],
                )

        @pl.when(c == 0)
        def _():
            _wout(p0_hbm)

        @pl.when(c == 1)
        def _():
            _wout(p1_hbm)

    return hist(values2d, count)


def _merge_tc(p0, p1):
    blk = 102_400

    def mk(a_ref, b_ref, o_ref):
        o_ref[...] = a_ref[...] + b_ref[...]

    return pl.pallas_call(
        mk,
        out_shape=jax.ShapeDtypeStruct((_V,), jnp.float32),
        grid=(pl.cdiv(_V, blk),),
        in_specs=[pl.BlockSpec((blk,), lambda i: (i,))] * 2,
        out_specs=pl.BlockSpec((blk,), lambda i: (i,)),
    )(p0, p1)


def kernel(values, lengths, count):
    vout, p0, p1 = _hist_sc(values, count)
    new_count = _merge_tc(p0, p1)
    return (vout, lengths, new_count)
